# baseline (device time: 16236 ns/iter reference)
import jax
import jax.numpy as jnp
from jax import lax
from jax.experimental import pallas as pl
from jax.experimental.pallas import tpu as pltpu

N_DEV = 4


def kernel(partial, gamma):
    x = partial.reshape(partial.shape[1], partial.shape[2])
    g = gamma.reshape(1, -1)
    m_tot, d = x.shape
    m_per = m_tot // N_DEV
    dh = d // 2

    def body(
        x_ref, g_ref, out_ref,
        send_a1, send_b1, recv_a1, recv_b1,
        send_a2, send_b2, recv_a2, recv_b2,
        loc_a2, loc_b2, part_a, part_b,
        send_sems, recv_sems,
    ):
        my = lax.axis_index("i")
        q1 = my ^ 1
        q3 = my ^ 3
        c1 = my ^ 1
        c2 = my ^ 2
        c3 = my ^ 3

        def row(c):
            return pl.ds(c * m_per, m_per)

        def copy(src, dst, sem, target):
            return pltpu.make_async_remote_copy(
                src_ref=src, dst_ref=dst,
                send_sem=send_sems.at[sem], recv_sem=recv_sems.at[sem],
                device_id=(target,), device_id_type=pl.DeviceIdType.MESH,
            )

        barrier_sem = pltpu.get_barrier_semaphore()
        for nbr in (q1, q3):
            pl.semaphore_signal(
                barrier_sem, inc=1,
                device_id=(nbr,), device_id_type=pl.DeviceIdType.MESH,
            )

        send_a1[0, :, :] = x_ref[row(c2), :dh].astype(jnp.bfloat16)
        send_b1[0, :, :] = x_ref[row(c2), dh:].astype(jnp.bfloat16)

        pl.semaphore_wait(barrier_sem, 2)

        rdma_a1u = copy(send_a1.at[0], recv_a1.at[0], 0, q1)
        rdma_b1u = copy(send_b1.at[0], recv_b1.at[0], 1, q3)
        rdma_a1u.start()
        rdma_b1u.start()

        send_a1[1, :, :] = x_ref[row(c1), :dh].astype(jnp.bfloat16)
        send_b1[1, :, :] = x_ref[row(c3), dh:].astype(jnp.bfloat16)
        rdma_a1l = copy(send_a1.at[1], recv_a1.at[1], 2, q1)
        rdma_b1l = copy(send_b1.at[1], recv_b1.at[1], 3, q3)
        rdma_a1l.start()
        rdma_b1l.start()

        loc_a2[:, :] = x_ref[row(c3), :dh].astype(jnp.bfloat16)
        loc_b2[:, :] = x_ref[row(c1), dh:].astype(jnp.bfloat16)

        rdma_a1u.wait_recv()
        send_a2[:, :] = loc_a2[:, :] + recv_a1[0, :, :]
        rdma_a2 = copy(send_a2, recv_a2, 4, q3)
        rdma_a2.start()

        rdma_b1u.wait_recv()
        send_b2[:, :] = loc_b2[:, :] + recv_b1[0, :, :]
        rdma_b2 = copy(send_b2, recv_b2, 5, q1)
        rdma_b2.start()

        rdma_a1l.wait_recv()
        part_a[:, :] = x_ref[row(my), :dh] + recv_a1[1, :, :].astype(
            jnp.float32
        )
        rdma_b1l.wait_recv()
        part_b[:, :] = x_ref[row(my), dh:] + recv_b1[1, :, :].astype(
            jnp.float32
        )

        rdma_a2.wait_recv()
        y_a = part_a[:, :] + recv_a2[:, :].astype(jnp.float32)
        ss_a = jnp.sum(y_a * y_a, axis=-1, keepdims=True)
        rdma_b2.wait_recv()
        y_b = part_b[:, :] + recv_b2[:, :].astype(jnp.float32)
        ss = ss_a + jnp.sum(y_b * y_b, axis=-1, keepdims=True)
        inv_rms = lax.rsqrt(ss / d + 1e-6)
        out_ref[:, :dh] = y_a * inv_rms * g_ref[0, :dh]
        out_ref[:, dh:] = y_b * inv_rms * g_ref[0, dh:]

        for r in (rdma_a1u, rdma_b1u, rdma_a1l, rdma_b1l, rdma_a2, rdma_b2):
            r.wait_send()

    bf = jnp.bfloat16
    return pl.pallas_call(
        body,
        out_shape=jax.ShapeDtypeStruct((m_per, d), jnp.float32),
        in_specs=[
            pl.BlockSpec(memory_space=pltpu.VMEM),
            pl.BlockSpec(memory_space=pltpu.VMEM),
        ],
        out_specs=pl.BlockSpec(memory_space=pltpu.VMEM),
        scratch_shapes=[
            pltpu.VMEM((2, m_per, dh), bf),
            pltpu.VMEM((2, m_per, dh), bf),
            pltpu.VMEM((2, m_per, dh), bf),
            pltpu.VMEM((2, m_per, dh), bf),
            pltpu.VMEM((m_per, dh), bf),
            pltpu.VMEM((m_per, dh), bf),
            pltpu.VMEM((m_per, dh), bf),
            pltpu.VMEM((m_per, dh), bf),
            pltpu.VMEM((m_per, dh), bf),
            pltpu.VMEM((m_per, dh), bf),
            pltpu.VMEM((m_per, dh), jnp.float32),
            pltpu.VMEM((m_per, dh), jnp.float32),
            pltpu.SemaphoreType.DMA((6,)),
            pltpu.SemaphoreType.DMA((6,)),
        ],
        compiler_params=pltpu.CompilerParams(collective_id=0),
    )(x, g)
